# Initial kernel scaffold; baseline (speedup 1.0000x reference)
import jax
import jax.numpy as jnp
from jax.experimental import pallas as pl


def kernel(inp, W):
    raise NotImplementedError("write your pallas kernel here")


# speedup vs baseline: 2.6120x; 2.6120x over previous
import jax
import jax.numpy as jnp
from jax import lax
from jax.experimental import pallas as pl


def _som_body(inp_ref, w_ref, out_ref):
    x = inp_ref[0]
    w = w_ref[...]
    xt = x.T
    K = w.shape[0]
    T = xt.shape[0]
    x_norm = jnp.sum(xt * xt, axis=1, keepdims=True)
    w_norm = jnp.sum(w * w, axis=1)
    dots = lax.dot_general(xt, w, (((1,), (1,)), ((), ())),
                           preferred_element_type=jnp.float32)
    dist = (x_norm + w_norm[None, :]) - 2.0 * dots
    m = jnp.min(dist, axis=1, keepdims=True)
    lane_k = lax.broadcasted_iota(jnp.int32, (T, K), 1)
    kidx = jnp.min(jnp.where(dist == m, lane_k, K), axis=1,
                   keepdims=True)
    kidx_row = kidx.T
    sub_k = lax.broadcasted_iota(jnp.int32, (K, T), 0)
    out_ref[0, 0] = (sub_k == kidx_row).astype(jnp.float32)


def kernel(inp, W):
    B, C, T = inp.shape
    K = W.shape[0]
    return pl.pallas_call(
        _som_body,
        grid=(B,),
        in_specs=[
            pl.BlockSpec((1, C, T), lambda b: (b, 0, 0)),
            pl.BlockSpec((K, C), lambda b: (0, 0)),
        ],
        out_specs=pl.BlockSpec((1, 1, K, T), lambda b: (b, 0, 0, 0)),
        out_shape=jax.ShapeDtypeStruct((B, 1, K, T), jnp.float32),
    )(inp, W)
